# baseline (speedup 1.0000x reference)
import jax
import jax.numpy as jnp
from jax import lax
from jax.experimental import pallas as pl
from jax.experimental.pallas import tpu as pltpu
from jax.experimental.pallas import tpu_sc as plsc

NC = 2
NS = 16
NW = NC * NS
L = 16
B = 65536
BPW = B // NW
NVEC = BPW // L
NSIG = 1000
BIG = 3.0e37


def _sigma_to_t_body(sigma_hbm, sigmas_hbm, out_hbm,
                     tab_v, q_v, o_v, sem_a, sem_b):
    wid = lax.axis_index("s") * NC + lax.axis_index("c")
    base = wid * BPW
    cp_tab = pltpu.async_copy(sigmas_hbm, tab_v, sem_a)
    cp_q = pltpu.async_copy(sigma_hbm.at[pl.ds(base, BPW)], q_v, sem_b)
    cp_tab.wait()
    cp_q.wait()

    def run_range(i0, i1):
        @plsc.parallel_loop(i0, i1, unroll=1)
        def step(i):
            q = q_v[pl.ds(i * L, L)]
            im = jnp.full((L,), -1, jnp.int32)
            for s in (512, 256, 128, 64, 32, 16, 8, 4, 2, 1):
                probe = jnp.minimum(im + s, NSIG - 1)
                v = plsc.load_gather(tab_v, [probe])
                im = jnp.where(v < q, probe, im)
            c = im + 1

            vb1 = plsc.load_gather(tab_v, [jnp.maximum(im, 0)])
            va1 = plsc.load_gather(tab_v, [jnp.minimum(c, NSIG - 1)])
            db1 = jnp.where(im >= 0, q - vb1, BIG)
            da1 = jnp.where(c <= NSIG - 1, va1 - q, BIG)
            nb = db1 <= da1

            i3 = jnp.where(nb, im - 1, im + 2)
            v3 = plsc.load_gather(tab_v, [jnp.clip(i3, 0, NSIG - 1)])
            bad3 = jnp.where(nb, im < 1, im > NSIG - 3)
            d3 = jnp.where(bad3, BIG, jnp.abs(v3 - q))
            d_o = jnp.where(nb, da1, db1)
            pick3 = jnp.where(nb, d3 <= d_o, d3 < d_o)

            w0 = jnp.clip((vb1 - q) / (vb1 - va1), 0.0, 1.0)
            w = jnp.where(pick3, jnp.where(nb, 0.0, 1.0), w0)
            t = (1.0 - w) * im.astype(jnp.float32) \
                + w * c.astype(jnp.float32)
            o_v[pl.ds(i * L, L)] = t

    run_range(0, NVEC)
    pltpu.sync_copy(o_v, out_hbm.at[pl.ds(base, BPW)])


@jax.jit
def kernel(sigma, sigmas):
    mesh = plsc.VectorSubcoreMesh(core_axis_name="c", subcore_axis_name="s")
    run = pl.kernel(
        _sigma_to_t_body,
        mesh=mesh,
        out_type=jax.ShapeDtypeStruct((B,), jnp.float32),
        scratch_types=[
            pltpu.VMEM((NSIG,), jnp.float32),
            pltpu.VMEM((BPW,), jnp.float32),
            pltpu.VMEM((BPW,), jnp.float32),
            pltpu.SemaphoreType.DMA,
            pltpu.SemaphoreType.DMA,
        ],
        compiler_params=pltpu.CompilerParams(needs_layout_passes=False),
    )
    return run(sigma, sigmas)


# speedup vs baseline: 1.1864x; 1.1864x over previous
import jax
import jax.numpy as jnp
from jax import lax
from jax.experimental import pallas as pl
from jax.experimental.pallas import tpu as pltpu
from jax.experimental.pallas import tpu_sc as plsc

NC = 2
NS = 16
NW = NC * NS
L = 16
B = 65536
BPW = B // NW
NVEC = BPW // L
NSIG = 1000
NPAD = 1024
BIG = 3.0e37


def _sigma_to_t_body(sigma_hbm, sigmas_hbm, out_hbm,
                     tab_v, q_v, o_v, sem_a, sem_b):
    wid = lax.axis_index("s") * NC + lax.axis_index("c")
    base = wid * BPW
    cp_tab = pltpu.async_copy(sigmas_hbm, tab_v.at[pl.ds(0, NSIG)], sem_a)
    cp_q = pltpu.async_copy(sigma_hbm.at[pl.ds(base, BPW)], q_v, sem_b)
    cp_tab.wait()
    cp_q.wait()

    def run_range(i0, i1):
        @plsc.parallel_loop(i0, i1, unroll=1)
        def step(i):
            q = q_v[pl.ds(i * L, L)]
            lo = jnp.zeros((L,), jnp.int32)
            hi = jnp.full((L,), NSIG, jnp.int32)
            for _ in range(10):
                mid = (lo + hi) >> 1
                v = plsc.load_gather(tab_v, [mid])
                pred = v < q
                lo = jnp.where(pred, mid + 1, lo)
                hi = jnp.where(pred, hi, mid)
            c = hi
            im = c - 1

            vb1 = plsc.load_gather(tab_v, [jnp.maximum(im, 0)])
            va1 = plsc.load_gather(tab_v, [c])
            db1 = jnp.where(im >= 0, q - vb1, BIG)
            da1 = jnp.where(c <= NSIG - 1, va1 - q, BIG)
            nb = db1 <= da1

            i3 = jnp.where(nb, im - 1, im + 2)
            v3 = plsc.load_gather(tab_v, [jnp.maximum(i3, 0)])
            bad3 = jnp.where(nb, im < 1, im > NSIG - 3)
            d3 = jnp.where(bad3, BIG, jnp.abs(v3 - q))
            d_o = jnp.where(nb, da1, db1)
            pick3 = jnp.where(nb, d3 <= d_o, d3 < d_o)

            w0 = jnp.clip((vb1 - q) / (vb1 - va1), 0.0, 1.0)
            w = jnp.where(pick3, jnp.where(nb, 0.0, 1.0), w0)
            t = (1.0 - w) * im.astype(jnp.float32) \
                + w * c.astype(jnp.float32)
            o_v[pl.ds(i * L, L)] = t

    run_range(0, NVEC)
    pltpu.sync_copy(o_v, out_hbm.at[pl.ds(base, BPW)])


@jax.jit
def kernel(sigma, sigmas):
    mesh = plsc.VectorSubcoreMesh(core_axis_name="c", subcore_axis_name="s")
    run = pl.kernel(
        _sigma_to_t_body,
        mesh=mesh,
        out_type=jax.ShapeDtypeStruct((B,), jnp.float32),
        scratch_types=[
            pltpu.VMEM((NPAD,), jnp.float32),
            pltpu.VMEM((BPW,), jnp.float32),
            pltpu.VMEM((BPW,), jnp.float32),
            pltpu.SemaphoreType.DMA,
            pltpu.SemaphoreType.DMA,
        ],
        compiler_params=pltpu.CompilerParams(needs_layout_passes=False),
    )
    return run(sigma, sigmas)
